# baseline (speedup 1.0000x reference)
import functools

import jax
import jax.numpy as jnp
from jax import lax
from jax.experimental import pallas as pl
from jax.experimental.pallas import tpu as pltpu
from jax.experimental.pallas import tpu_sc as plsc

N_NUM = 10
N_CAT = 26
N_PAD = 32
N_TOT = N_CAT + N_NUM
B = 4096
D = 128
VOCAB = 1000

NC = 2
NS = 16
NW = NC * NS
BPW = B // NW

N_BUF = 6
LOOKAHEAD = 4

_mesh = plsc.VectorSubcoreMesh(
    core_axis_name="c", subcore_axis_name="s", num_cores=NC, num_subcores=NS
)


@functools.partial(
    pl.kernel,
    out_type=jax.ShapeDtypeStruct((B, N_PAD, D), jnp.float32),
    mesh=_mesh,
    scratch_types=(
        [pltpu.VMEM((N_CAT, BPW), jnp.int32)]
        + [pltpu.VMEM((BPW, D), jnp.float32) for _ in range(N_BUF)]
        + [pltpu.SemaphoreType.DMA for _ in range(2 * N_BUF)]
    ),
)
def _gather_kernel(tables, idx, out, idx_v, *bufs_and_sems):
    gbufs = bufs_and_sems[:N_BUF]
    gsems = bufs_and_sems[N_BUF:2 * N_BUF]
    wsems = bufs_and_sems[2 * N_BUF:]

    wid = lax.axis_index("s") * NC + lax.axis_index("c")
    b0 = wid * BPW

    pltpu.sync_copy(idx.at[:, pl.ds(b0, BPW)], idx_v)

    def gather(f):
        return pltpu.async_copy(
            tables.at[idx_v.at[f]], gbufs[f % N_BUF], gsems[f % N_BUF]
        )

    gc = [None] * N_CAT
    wc = [None] * N_CAT
    waited = set()
    for f in range(LOOKAHEAD):
        gc[f] = gather(f)
    for f in range(N_CAT):
        nf = f + LOOKAHEAD
        if nf < N_CAT:
            if nf - N_BUF >= 0:
                wc[nf - N_BUF].wait()
                waited.add(nf - N_BUF)
            gc[nf] = gather(nf)
        gc[f].wait()
        wc[f] = pltpu.async_copy(
            gbufs[f % N_BUF], out.at[pl.ds(b0, BPW), f], wsems[f % N_BUF]
        )
    for f in range(N_CAT):
        if f not in waited:
            wc[f].wait()


BM = 256


def _finalize_body(cat_ref, x_ref, w_ref, o_ref):
    o_ref[:, :N_PAD, :] = cat_ref[...]
    for j in range(N_NUM):
        o_ref[:, N_CAT + j, :] = x_ref[j] * w_ref[j]


_finalize = pl.pallas_call(
    _finalize_body,
    out_shape=jax.ShapeDtypeStruct((B, N_TOT, D), jnp.float32),
    grid=(B // BM,),
    in_specs=[
        pl.BlockSpec((BM, N_PAD, D), lambda m: (m, 0, 0)),
        pl.BlockSpec((N_NUM, BM, 1), lambda m: (0, m, 0)),
        pl.BlockSpec((N_NUM, 1, D), lambda m: (0, 0, 0)),
    ],
    out_specs=pl.BlockSpec((BM, N_TOT, D), lambda m: (m, 0, 0)),
)


def kernel(num_features, cat_features, cat_tables, num_weights):
    flat_tables = cat_tables.reshape(N_CAT * (VOCAB + 1), D)
    offs = (jnp.arange(N_CAT, dtype=jnp.int32) * (VOCAB + 1))[:, None]
    idx = cat_features.astype(jnp.int32) + offs
    cat3 = _gather_kernel(flat_tables, idx)
    return _finalize(cat3, num_features, num_weights)


# speedup vs baseline: 2.1282x; 2.1282x over previous
import functools

import jax
import jax.numpy as jnp
from jax import lax
from jax.experimental import pallas as pl
from jax.experimental.pallas import tpu as pltpu
from jax.experimental.pallas import tpu_sc as plsc

N_NUM = 10
N_CAT = 26
N_TOT = N_CAT + N_NUM
B = 4096
D = 128
VOCAB = 1000

NC = 2
NS = 16
NW = NC * NS
BPW = B // NW

N_BUF = 5
LOOKAHEAD = 3

_mesh = plsc.VectorSubcoreMesh(
    core_axis_name="c", subcore_axis_name="s", num_cores=NC, num_subcores=NS
)


@functools.partial(
    pl.kernel,
    out_type=jax.ShapeDtypeStruct((N_TOT, B, D), jnp.float32),
    mesh=_mesh,
    scratch_types=(
        [
            pltpu.VMEM((N_CAT, BPW), jnp.int32),
            pltpu.VMEM((N_NUM, BPW), jnp.float32),
            pltpu.VMEM((N_NUM, D), jnp.float32),
        ]
        + [pltpu.VMEM((BPW, D), jnp.float32) for _ in range(N_BUF + 2)]
        + [pltpu.SemaphoreType.DMA for _ in range(2 * N_BUF + 2)]
    ),
)
def _emb_kernel(tables, idx, xs, ws, out, idx_v, x_v, w_v, *rest):
    gbufs = rest[:N_BUF]
    nbufs = rest[N_BUF:N_BUF + 2]
    gsems = rest[N_BUF + 2:2 * N_BUF + 2]
    wsems = rest[2 * N_BUF + 2:3 * N_BUF + 2]
    nsems = rest[3 * N_BUF + 2:]

    wid = lax.axis_index("s") * NC + lax.axis_index("c")
    b0 = wid * BPW

    pltpu.sync_copy(idx.at[:, pl.ds(b0, BPW)], idx_v)
    pltpu.sync_copy(xs.at[:, pl.ds(b0, BPW)], x_v)
    pltpu.sync_copy(ws, w_v)

    def gather(f):
        return pltpu.async_copy(
            tables.at[idx_v.at[f]], gbufs[f % N_BUF], gsems[f % N_BUF]
        )

    gc = [None] * N_CAT
    wc = [None] * N_CAT
    nwc = [None] * N_NUM
    waited = set()

    def num_col(j):
        if j >= 2:
            nwc[j - 2].wait()
        nb = nbufs[j % 2]
        wregs = [w_v[j, pl.ds(r * 16, 16)] for r in range(D // 16)]

        def body(g, _):
            xv = x_v[j, pl.ds(g * 16, 16)]
            for l in range(16):
                x = xv[l]
                for r in range(D // 16):
                    nb[g * 16 + l, pl.ds(r * 16, 16)] = x * wregs[r]
            return ()

        lax.fori_loop(0, BPW // 16, body, ())
        nwc[j] = pltpu.async_copy(
            nb, out.at[N_CAT + j, pl.ds(b0, BPW)], nsems[j % 2]
        )

    for f in range(LOOKAHEAD):
        gc[f] = gather(f)
    for f in range(N_CAT):
        nf = f + LOOKAHEAD
        if nf < N_CAT:
            if nf - N_BUF >= 0:
                wc[nf - N_BUF].wait()
                waited.add(nf - N_BUF)
            gc[nf] = gather(nf)
        if f < N_NUM:
            num_col(f)
        gc[f].wait()
        wc[f] = pltpu.async_copy(
            gbufs[f % N_BUF], out.at[f, pl.ds(b0, BPW)], wsems[f % N_BUF]
        )
    for f in range(N_CAT):
        if f not in waited:
            wc[f].wait()
    nwc[N_NUM - 2].wait()
    nwc[N_NUM - 1].wait()


def kernel(num_features, cat_features, cat_tables, num_weights):
    flat_tables = cat_tables.reshape(N_CAT * (VOCAB + 1), D)
    offs = (jnp.arange(N_CAT, dtype=jnp.int32) * (VOCAB + 1))[:, None]
    idx = cat_features.astype(jnp.int32) + offs
    xs = num_features.reshape(N_NUM, B)
    ws = num_weights.reshape(N_NUM, D)
    out = _emb_kernel(flat_tables, idx, xs, ws)
    return jnp.transpose(out, (1, 0, 2))


# speedup vs baseline: 2.2993x; 1.0804x over previous
import functools

import jax
import jax.numpy as jnp
from jax import lax
from jax.experimental import pallas as pl
from jax.experimental.pallas import tpu as pltpu
from jax.experimental.pallas import tpu_sc as plsc

N_NUM = 10
N_CAT = 26
N_TOT = N_CAT + N_NUM
B = 4096
D = 128
VOCAB = 1000

NC = 2
NS = 16
NW = NC * NS
BPW = B // NW

N_BUF = 5
LOOKAHEAD = 3

_mesh = plsc.VectorSubcoreMesh(
    core_axis_name="c", subcore_axis_name="s", num_cores=NC, num_subcores=NS
)


@functools.partial(
    pl.kernel,
    out_type=jax.ShapeDtypeStruct((N_TOT, B, D), jnp.float32),
    mesh=_mesh,
    scratch_types=(
        [
            pltpu.VMEM((N_CAT, BPW), jnp.int32),
            pltpu.VMEM((N_NUM, BPW), jnp.float32),
            pltpu.VMEM((N_NUM * D,), jnp.float32),
        ]
        + [pltpu.VMEM((BPW, D), jnp.float32) for _ in range(N_BUF + 2)]
        + [pltpu.SemaphoreType.DMA for _ in range(2 * N_BUF + 4)]
    ),
)
def _emb_kernel(tables, idx, xs, ws, out, idx_v, x_v, w_v, *rest):
    gbufs = rest[:N_BUF]
    nbufs = rest[N_BUF:N_BUF + 2]
    gsems = rest[N_BUF + 2:2 * N_BUF + 2]
    wsems = rest[2 * N_BUF + 2:3 * N_BUF + 2]
    nsems = rest[3 * N_BUF + 2:3 * N_BUF + 4]
    xsem = rest[3 * N_BUF + 4]

    wid = lax.axis_index("s") * NC + lax.axis_index("c")
    b0 = wid * BPW

    pltpu.sync_copy(idx.at[:, pl.ds(b0, BPW)], idx_v)

    def gather(f):
        return pltpu.async_copy(
            tables.at[f].at[idx_v.at[f]], gbufs[f % N_BUF], gsems[f % N_BUF]
        )

    gc = [None] * N_CAT
    wc = [None] * N_CAT
    nwc = [None] * N_NUM
    waited = set()

    for f in range(LOOKAHEAD):
        gc[f] = gather(f)

    xc = [
        pltpu.async_copy(xs.at[pl.ds(j * B + b0, BPW)], x_v.at[j], xsem)
        for j in range(N_NUM)
    ]
    pltpu.sync_copy(ws, w_v)

    def num_col(j):
        if j == 0:
            for c in xc:
                c.wait()
        if j >= 2:
            nwc[j - 2].wait()
        nb = nbufs[j % 2]
        wregs = [w_v[pl.ds(j * D + r * 16, 16)] for r in range(D // 16)]

        def body(g, _):
            xv = x_v[j, pl.ds(g * 16, 16)]
            for l in range(16):
                x = xv[l]
                for r in range(D // 16):
                    nb[g * 16 + l, pl.ds(r * 16, 16)] = x * wregs[r]
            return ()

        lax.fori_loop(0, BPW // 16, body, ())
        nwc[j] = pltpu.async_copy(
            nb, out.at[N_CAT + j, pl.ds(b0, BPW)], nsems[j % 2]
        )

    for f in range(N_CAT):
        nf = f + LOOKAHEAD
        if nf < N_CAT:
            if nf - N_BUF >= 0:
                wc[nf - N_BUF].wait()
                waited.add(nf - N_BUF)
            gc[nf] = gather(nf)
        if f < N_NUM:
            num_col(f)
        gc[f].wait()
        wc[f] = pltpu.async_copy(
            gbufs[f % N_BUF], out.at[f, pl.ds(b0, BPW)], wsems[f % N_BUF]
        )
    for f in range(N_CAT):
        if f not in waited:
            wc[f].wait()
    nwc[N_NUM - 2].wait()
    nwc[N_NUM - 1].wait()


def kernel(num_features, cat_features, cat_tables, num_weights):
    idx = cat_features.astype(jnp.int32)
    xs = num_features.reshape(N_NUM * B)
    ws = num_weights.reshape(N_NUM * D)
    out = _emb_kernel(cat_tables, idx, xs, ws)
    return jnp.transpose(out, (1, 0, 2))


# speedup vs baseline: 2.5016x; 1.0880x over previous
import functools

import jax
import jax.numpy as jnp
from jax import lax
from jax.experimental import pallas as pl
from jax.experimental.pallas import tpu as pltpu
from jax.experimental.pallas import tpu_sc as plsc

N_NUM = 10
N_CAT = 26
N_TOT = N_CAT + N_NUM
B = 4096
D = 128
VOCAB = 1000

NC = 2
NS = 16
NW = NC * NS
BPW = B // NW

N_BUF = 7
LOOKAHEAD = 5

_mesh = plsc.VectorSubcoreMesh(
    core_axis_name="c", subcore_axis_name="s", num_cores=NC, num_subcores=NS
)


@functools.partial(
    pl.kernel,
    out_type=jax.ShapeDtypeStruct((N_TOT, B, D), jnp.float32),
    mesh=_mesh,
    scratch_types=(
        [pltpu.VMEM((N_CAT, BPW), jnp.int32)]
        + [pltpu.VMEM((BPW, D), jnp.float32) for _ in range(N_BUF)]
        + [pltpu.SemaphoreType.DMA for _ in range(2 * N_BUF)]
    ),
)
def _gather_kernel(tables, idx, out, idx_v, *rest):
    gbufs = rest[:N_BUF]
    gsems = rest[N_BUF:2 * N_BUF]
    wsems = rest[2 * N_BUF:]

    wid = lax.axis_index("s") * NC + lax.axis_index("c")
    b0 = wid * BPW

    pltpu.sync_copy(idx.at[:, pl.ds(b0, BPW)], idx_v)

    def gather(f):
        return pltpu.async_copy(
            tables.at[f].at[idx_v.at[f]], gbufs[f % N_BUF], gsems[f % N_BUF]
        )

    gc = [None] * N_CAT
    wc = [None] * N_CAT
    waited = set()
    for f in range(LOOKAHEAD):
        gc[f] = gather(f)
    for f in range(N_CAT):
        nf = f + LOOKAHEAD
        if nf < N_CAT:
            if nf - N_BUF >= 0:
                wc[nf - N_BUF].wait()
                waited.add(nf - N_BUF)
            gc[nf] = gather(nf)
        gc[f].wait()
        wc[f] = pltpu.async_copy(
            gbufs[f % N_BUF], out.at[f, pl.ds(b0, BPW)], wsems[f % N_BUF]
        )
    for f in range(N_CAT):
        if f not in waited:
            wc[f].wait()


def _num_body(x_ref, w_ref, cat_ref, o_ref):
    del cat_ref
    o_ref[0] = x_ref[...][:, None] * w_ref[...][None, :]


_num_call = pl.pallas_call(
    _num_body,
    out_shape=jax.ShapeDtypeStruct((N_TOT, B, D), jnp.float32),
    grid=(N_NUM,),
    in_specs=[
        pl.BlockSpec((B,), lambda j: (j,)),
        pl.BlockSpec((D,), lambda j: (j,)),
        pl.BlockSpec((1, 8, D), lambda j: (0, 0, 0)),
    ],
    out_specs=pl.BlockSpec((1, B, D), lambda j: (N_CAT + j, 0, 0)),
    input_output_aliases={2: 0},
)


def kernel(num_features, cat_features, cat_tables, num_weights):
    idx = cat_features.astype(jnp.int32)
    xs = num_features.reshape(N_NUM * B)
    ws = num_weights.reshape(N_NUM * D)
    cat = _gather_kernel(cat_tables, idx)
    out = _num_call(xs, ws, cat)
    return jnp.transpose(out, (1, 0, 2))
